# baseline (speedup 1.0000x reference)
import functools

import jax
import jax.numpy as jnp
from jax import lax
from jax.experimental import pallas as pl
from jax.experimental.pallas import tpu as pltpu
from jax.experimental.pallas import tpu_sc as plsc

BATCH = 16384
EMB = 64

_NC = 2
_NS = 16
_NW = _NC * _NS
_BPW = BATCH // _NW
_CH = 128
_NCHUNK = _BPW // _CH


def _sc_gather(uid3, iid3, gmf_user, gmf_item, mlp_user, mlp_item):
    mesh = plsc.VectorSubcoreMesh(core_axis_name="c", subcore_axis_name="s")
    out_t = [jax.ShapeDtypeStruct((BATCH, EMB), jnp.float32)] * 4

    @functools.partial(
        pl.kernel,
        out_type=out_t,
        mesh=mesh,
        scratch_types=[
            pltpu.VMEM((_NCHUNK, _CH), jnp.int32),
            pltpu.VMEM((_NCHUNK, _CH), jnp.int32),
            pltpu.VMEM((_BPW, EMB), jnp.float32),
            pltpu.VMEM((_BPW, EMB), jnp.float32),
            pltpu.SemaphoreType.DMA,
            pltpu.SemaphoreType.DMA,
        ],
        compiler_params=pltpu.CompilerParams(use_tc_tiling_on_sc=False),
    )
    def sc_k(uid_h, iid_h, gu_h, gi_h, mu_h, mi_h, ogu, ogi, omu, omi,
             uv, iv, rows0, rows1, s0, s1):
        wid = lax.axis_index("s") * _NC + lax.axis_index("c")
        base = wid * _BPW
        pltpu.sync_copy(uid_h.at[wid], uv)
        pltpu.sync_copy(iid_h.at[wid], iv)

        def fire(table, idxv, buf, sem):
            cs = []
            for j in range(_NCHUNK):
                cs.append(pltpu.async_copy(
                    table.at[idxv.at[j]], buf.at[pl.ds(j * _CH, _CH)], sem))
            return cs

        def drain(cs):
            for c in cs:
                c.wait()

        c0 = fire(gu_h, uv, rows0, s0)
        c1 = fire(gi_h, iv, rows1, s1)
        drain(c0)
        pltpu.sync_copy(rows0, ogu.at[pl.ds(base, _BPW)])
        c2 = fire(mu_h, uv, rows0, s0)
        drain(c1)
        pltpu.sync_copy(rows1, ogi.at[pl.ds(base, _BPW)])
        c3 = fire(mi_h, iv, rows1, s1)
        drain(c2)
        pltpu.sync_copy(rows0, omu.at[pl.ds(base, _BPW)])
        drain(c3)
        pltpu.sync_copy(rows1, omi.at[pl.ds(base, _BPW)])

    return sc_k(uid3, iid3, gmf_user, gmf_item, mlp_user, mlp_item)


_BB = 2048


def _tc_body(gu, gi, mu, mi, w1u, w1i, b1, w2, b2, w3, b3, wog, woh, bo, out):
    h = jnp.dot(mu[...], w1u[...], preferred_element_type=jnp.float32)
    h = h + jnp.dot(mi[...], w1i[...], preferred_element_type=jnp.float32)
    h = jnp.maximum(h + b1[...], 0.0)
    h = jnp.maximum(
        jnp.dot(h, w2[...], preferred_element_type=jnp.float32) + b2[...], 0.0)
    h = jnp.maximum(
        jnp.dot(h, w3[...], preferred_element_type=jnp.float32) + b3[...], 0.0)
    g = gu[...] * gi[...]
    p = jnp.dot(g, wog[...], preferred_element_type=jnp.float32)
    p = p + jnp.dot(h, woh[...], preferred_element_type=jnp.float32)
    out[...] = p + bo[...]


def _tc_mlp(gu, gi, mu, mi, w1u, w1i, b1, w2, b2, w3, b3, wog, woh, bo):
    grid = (BATCH // _BB,)
    fixed = lambda i: (0, 0)
    in_specs = [
        pl.BlockSpec((_BB, EMB), lambda i: (i, 0)),
        pl.BlockSpec((_BB, EMB), lambda i: (i, 0)),
        pl.BlockSpec((_BB, EMB), lambda i: (i, 0)),
        pl.BlockSpec((_BB, EMB), lambda i: (i, 0)),
        pl.BlockSpec(w1u.shape, fixed),
        pl.BlockSpec(w1i.shape, fixed),
        pl.BlockSpec(b1.shape, fixed),
        pl.BlockSpec(w2.shape, fixed),
        pl.BlockSpec(b2.shape, fixed),
        pl.BlockSpec(w3.shape, fixed),
        pl.BlockSpec(b3.shape, fixed),
        pl.BlockSpec(wog.shape, fixed),
        pl.BlockSpec(woh.shape, fixed),
        pl.BlockSpec(bo.shape, fixed),
    ]
    return pl.pallas_call(
        _tc_body,
        grid=grid,
        in_specs=in_specs,
        out_specs=pl.BlockSpec((_BB, 1), lambda i: (i, 0)),
        out_shape=jax.ShapeDtypeStruct((BATCH, 1), jnp.float32),
        compiler_params=pltpu.CompilerParams(
            dimension_semantics=("parallel",)),
    )(gu, gi, mu, mi, w1u, w1i, b1, w2, b2, w3, b3, wog, woh, bo)


def kernel(user_ids, item_ids, gmf_user, gmf_item, mlp_user, mlp_item,
           W1, b1, W2, b2, W3, b3, Wo, bo):
    uid3 = user_ids.astype(jnp.int32).reshape(_NW, _NCHUNK, _CH)
    iid3 = item_ids.astype(jnp.int32).reshape(_NW, _NCHUNK, _CH)
    gu, gi, mu, mi = _sc_gather(uid3, iid3, gmf_user, gmf_item,
                                mlp_user, mlp_item)
    pred = _tc_mlp(
        gu, gi, mu, mi,
        W1[:EMB], W1[EMB:], b1.reshape(1, -1),
        W2, b2.reshape(1, -1),
        W3, b3.reshape(1, -1),
        Wo[:EMB], Wo[EMB:], bo.reshape(1, 1),
    )
    return pred[:, 0]


# speedup vs baseline: 1.2308x; 1.2308x over previous
import functools

import jax
import jax.numpy as jnp
from jax import lax
from jax.experimental import pallas as pl
from jax.experimental.pallas import tpu as pltpu
from jax.experimental.pallas import tpu_sc as plsc

BATCH = 16384
EMB = 64

_NC = 2
_NS = 16
_NW = _NC * _NS
_BPW = BATCH // _NW
_CH = 128
_NCHUNK = _BPW // _CH
_SUB = 2 * _CH


def _sc_gather(uid3, iid3, utab, itab):
    mesh = plsc.VectorSubcoreMesh(core_axis_name="c", subcore_axis_name="s")
    out_t = [jax.ShapeDtypeStruct((BATCH, 2 * EMB), jnp.float32)] * 2

    @functools.partial(
        pl.kernel,
        out_type=out_t,
        mesh=mesh,
        scratch_types=[
            pltpu.VMEM((_NCHUNK, _CH), jnp.int32),
            pltpu.VMEM((_NCHUNK, _CH), jnp.int32),
            pltpu.VMEM((_SUB, 2 * EMB), jnp.float32),
            pltpu.VMEM((_SUB, 2 * EMB), jnp.float32),
            pltpu.SemaphoreType.DMA,
            pltpu.SemaphoreType.DMA,
        ],
    )
    def sc_k(uid_h, iid_h, ut_h, it_h, ou, oi, uv, iv, rows0, rows1, s0, s1):
        wid = lax.axis_index("s") * _NC + lax.axis_index("c")
        base = wid * _BPW
        pltpu.sync_copy(uid_h.at[wid], uv)
        pltpu.sync_copy(iid_h.at[wid], iv)

        def fire(table, idxv, half, buf, sem):
            cs = []
            for j in range(2):
                cs.append(pltpu.async_copy(
                    table.at[idxv.at[2 * half + j]],
                    buf.at[pl.ds(j * _CH, _CH)], sem))
            return cs

        def drain(cs):
            for c in cs:
                c.wait()

        c0 = fire(ut_h, uv, 0, rows0, s0)
        c1 = fire(ut_h, uv, 1, rows1, s1)
        drain(c0)
        pltpu.sync_copy(rows0, ou.at[pl.ds(base, _SUB)])
        c2 = fire(it_h, iv, 0, rows0, s0)
        drain(c1)
        pltpu.sync_copy(rows1, ou.at[pl.ds(base + _SUB, _SUB)])
        c3 = fire(it_h, iv, 1, rows1, s1)
        drain(c2)
        pltpu.sync_copy(rows0, oi.at[pl.ds(base, _SUB)])
        drain(c3)
        pltpu.sync_copy(rows1, oi.at[pl.ds(base + _SUB, _SUB)])

    return sc_k(uid3, iid3, utab, itab)


_BB = 2048


def _tc_body(ru, ri, w1u, w1i, b1, w2, b2, w3, b3, wog, woh, bo, out):
    u = ru[...]
    v = ri[...]
    h = jnp.dot(u[:, EMB:], w1u[...], preferred_element_type=jnp.float32)
    h = h + jnp.dot(v[:, EMB:], w1i[...], preferred_element_type=jnp.float32)
    h = jnp.maximum(h + b1[...], 0.0)
    h = jnp.maximum(
        jnp.dot(h, w2[...], preferred_element_type=jnp.float32) + b2[...], 0.0)
    h = jnp.maximum(
        jnp.dot(h, w3[...], preferred_element_type=jnp.float32) + b3[...], 0.0)
    g = u[:, :EMB] * v[:, :EMB]
    p = jnp.dot(g, wog[...], preferred_element_type=jnp.float32)
    p = p + jnp.dot(h, woh[...], preferred_element_type=jnp.float32)
    out[...] = p + bo[...]


def _tc_mlp(ru, ri, w1u, w1i, b1, w2, b2, w3, b3, wog, woh, bo):
    grid = (BATCH // _BB,)
    fixed = lambda i: (0, 0)
    in_specs = [
        pl.BlockSpec((_BB, 2 * EMB), lambda i: (i, 0)),
        pl.BlockSpec((_BB, 2 * EMB), lambda i: (i, 0)),
        pl.BlockSpec(w1u.shape, fixed),
        pl.BlockSpec(w1i.shape, fixed),
        pl.BlockSpec(b1.shape, fixed),
        pl.BlockSpec(w2.shape, fixed),
        pl.BlockSpec(b2.shape, fixed),
        pl.BlockSpec(w3.shape, fixed),
        pl.BlockSpec(b3.shape, fixed),
        pl.BlockSpec(wog.shape, fixed),
        pl.BlockSpec(woh.shape, fixed),
        pl.BlockSpec(bo.shape, fixed),
    ]
    return pl.pallas_call(
        _tc_body,
        grid=grid,
        in_specs=in_specs,
        out_specs=pl.BlockSpec((_BB, 1), lambda i: (i, 0)),
        out_shape=jax.ShapeDtypeStruct((BATCH, 1), jnp.float32),
        compiler_params=pltpu.CompilerParams(
            dimension_semantics=("parallel",)),
    )(ru, ri, w1u, w1i, b1, w2, b2, w3, b3, wog, woh, bo)


def kernel(user_ids, item_ids, gmf_user, gmf_item, mlp_user, mlp_item,
           W1, b1, W2, b2, W3, b3, Wo, bo):
    uid3 = user_ids.astype(jnp.int32).reshape(_NW, _NCHUNK, _CH)
    iid3 = item_ids.astype(jnp.int32).reshape(_NW, _NCHUNK, _CH)
    utab = jnp.concatenate([gmf_user, mlp_user], axis=1)
    itab = jnp.concatenate([gmf_item, mlp_item], axis=1)
    ru, ri = _sc_gather(uid3, iid3, utab, itab)
    pred = _tc_mlp(
        ru, ri,
        W1[:EMB], W1[EMB:], b1.reshape(1, -1),
        W2, b2.reshape(1, -1),
        W3, b3.reshape(1, -1),
        Wo[:EMB], Wo[EMB:], bo.reshape(1, 1),
    )
    return pred[:, 0]
